# baseline (speedup 1.0000x reference)
import functools

import jax
import jax.numpy as jnp
from jax import lax
from jax.experimental import pallas as pl
from jax.experimental.pallas import tpu as pltpu
from jax.experimental.pallas import tpu_sc as plsc

B = 16384
V = 100001
D = 32
NC, NS = 2, 16
NW = NC * NS
NBLK_TOT = 782
NBLK = 25
X = NBLK * 128
WIN = 1024
CPKCAP = WIN + 32
ROWCAP = 16
DUMP = B
OUTH = B + 8

_sc_mesh = plsc.VectorSubcoreMesh(core_axis_name="c", subcore_axis_name="s")


NCHAIN = 4
Q = B // NCHAIN


def _scan0(idxv, cpks, o0, o1, iota16):
    def group(k, tots):
        new = []
        for q in range(NCHAIN):
            iv = idxv[pl.ds(q * Q + k * 16, 16)]
            blk = lax.shift_right_logical(iv, 7)
            m = (blk >= o0) & (blk < o1)
            pk = jnp.bitwise_or(lax.shift_left(iv, 15),
                                iota16 + (q * Q + k * 16))
            plsc.store_compressed(
                cpks[q].at[pl.ds(jnp.minimum(tots[q], WIN + 16), 16)],
                pk, mask=m)
            new.append(tots[q] + plsc.all_reduce_population_count(m)[0])
        return tuple(new)

    return lax.fori_loop(0, Q // 16, group, (0,) * NCHAIN, unroll=4)


def _scan(idxv, cpk, o0, o1, iota16, win_lo, p0):
    def group(k, carry):
        tot, ca = carry
        iv = idxv[pl.ds(p0 + k * 16, 16)]
        blk = lax.shift_right_logical(iv, 7)
        m = (blk >= o0) & (blk < o1)
        pos = iota16 + (p0 + k * 16)
        pk = jnp.bitwise_or(lax.shift_left(iv, 15), pos)
        wok = (tot + 16 > win_lo) & (tot < win_lo + WIN)
        am = m & wok
        plsc.store_compressed(cpk.at[pl.ds(ca, 16)], pk, mask=am)
        tot = tot + plsc.all_reduce_population_count(m)[0]
        ca = ca + plsc.all_reduce_population_count(am)[0]
        return tot, ca
    return lax.fori_loop(0, Q // 16, group, (0, 0), unroll=4)


def _extract(out_hbm, ca, sbase, vlin, cpk, rows_a, rows_b, posv,
             sem_s, iota16):
    nflush = (ca + (ROWCAP - 1)) // ROWCAP

    def flush(f, _):
        parity = f % 2

        def work(rows):
            @pl.when(f >= 2)
            def _():
                pltpu.make_async_copy(
                    out_hbm.at[pl.ds(0, ROWCAP)], rows, sem_s).wait()

            v0 = cpk[pl.ds(f * ROWCAP, 16)]
            il0 = jnp.clip(lax.shift_right_logical(v0, 15) - sbase, 0, X - 1)
            for e in range(ROWCAP):
                iloc = il0[e]
                cv = jnp.full((16,), iloc, jnp.int32)
                rows[e, pl.ds(0, 16)] = plsc.load_gather(vlin, [iota16, cv])
                rows[e, pl.ds(16, 16)] = plsc.load_gather(
                    vlin, [iota16 + 16, cv])
            gidx0 = f * ROWCAP + iota16
            posv[0, pl.ds(0, 16)] = jnp.where(
                gidx0 < ca, jnp.bitwise_and(v0, 32767), DUMP)
            pltpu.async_copy(rows, out_hbm.at[posv.at[0]], sem_s)

        @pl.when(parity == 0)
        def _():
            work(rows_a)

        @pl.when(parity == 1)
        def _():
            work(rows_b)

        return 0

    lax.fori_loop(0, nflush, flush, 0, unroll=False)

    @pl.when(nflush >= 1)
    def _():
        pltpu.make_async_copy(
            out_hbm.at[pl.ds(0, ROWCAP)], rows_a, sem_s).wait()

    @pl.when(nflush >= 2)
    def _():
        pltpu.make_async_copy(
            out_hbm.at[pl.ds(0, ROWCAP)], rows_b, sem_s).wait()


def _do_table(tT_hbm, idx_hbm, out_hbm, s_blk, o0, o1, vlin, idxv,
              cpks, rows_a, rows_b, posv, sem_t, sem_i, sem_s,
              iota16):
    stage = [
        pltpu.async_copy(
            tT_hbm.at[pl.ds(tj * 8, 8), pl.ds(s_blk * 128, X)],
            vlin.at[pl.ds(tj * 8, 8)], sem_t)
        for tj in range(4)
    ]
    pltpu.async_copy(idx_hbm, idxv, sem_i).wait()
    sbase = s_blk * 128

    tots = _scan0(idxv, cpks, o0, o1, iota16)
    for s in stage:
        s.wait()
    for q in range(NCHAIN):
        _extract(out_hbm, jnp.minimum(tots[q], WIN), sbase, vlin, cpks[q],
                 rows_a, rows_b, posv, sem_s, iota16)

    for q in range(NCHAIN):
        def more(r, cpk_h=cpks[q], p0_h=q * Q):
            _, ca_r = _scan(idxv, cpk_h, o0, o1, iota16, r * WIN, p0_h)
            _extract(out_hbm, ca_r, sbase, vlin, cpk_h, rows_a, rows_b,
                     posv, sem_s, iota16)
            return r + 1

        lax.while_loop(lambda r, t=tots[q]: r * WIN < t, more, 1)


@functools.partial(
    pl.kernel,
    out_type=(
        jax.ShapeDtypeStruct((OUTH, 128), jnp.float32),
        jax.ShapeDtypeStruct((OUTH, 128), jnp.float32),
    ),
    mesh=_sc_mesh,
    scratch_types=[
        pltpu.VMEM((D, X), jnp.float32),
        pltpu.VMEM((B,), jnp.int32),
        pltpu.VMEM((CPKCAP,), jnp.int32),
        pltpu.VMEM((CPKCAP,), jnp.int32),
        pltpu.VMEM((CPKCAP,), jnp.int32),
        pltpu.VMEM((CPKCAP,), jnp.int32),
        pltpu.VMEM((ROWCAP, 128), jnp.float32),
        pltpu.VMEM((ROWCAP, 128), jnp.float32),
        pltpu.VMEM((1, ROWCAP), jnp.int32),
        pltpu.SemaphoreType.DMA,
        pltpu.SemaphoreType.DMA,
        pltpu.SemaphoreType.DMA,
    ],
    compiler_params=pltpu.CompilerParams(needs_layout_passes=False),
)
def _sc_gather(uT_hbm, mT_hbm, uid_hbm, mid_hbm, ue_hbm, me_hbm,
               vlin, idxv, cpk0, cpk1, cpk2, cpk3, rows_a, rows_b, posv,
               sem_t, sem_i, sem_s):
    cpks = (cpk0, cpk1, cpk2, cpk3)
    wid = lax.axis_index("s") * NC + lax.axis_index("c")
    o0 = wid * NBLK
    o1 = o0 + NBLK
    s_blk = jnp.minimum(o0, NBLK_TOT - NBLK)
    iota16 = lax.iota(jnp.int32, 16)

    _do_table(uT_hbm, uid_hbm, ue_hbm, s_blk, o0, o1, vlin, idxv,
              cpks, rows_a, rows_b, posv, sem_t, sem_i, sem_s, iota16)
    _do_table(mT_hbm, mid_hbm, me_hbm, s_blk, o0, o1, vlin, idxv,
              cpks, rows_a, rows_b, posv, sem_t, sem_i, sem_s, iota16)


BLK = 4096


def _mlp_body(ue_ref, me_ref, w1u_ref, w1m_ref, b1_ref, w2_ref, b2_ref,
              w3_ref, b3_ref, out_ref):
    ue = ue_ref[...][:, :D]
    me = me_ref[...][:, :D]
    x1 = jnp.dot(ue, w1u_ref[...], preferred_element_type=jnp.float32)
    x2 = jnp.dot(me, w1m_ref[...], preferred_element_type=jnp.float32)
    h1 = jnp.maximum(x1 + x2 + b1_ref[...], 0.0)
    h2 = jnp.maximum(
        jnp.dot(h1, w2_ref[...], preferred_element_type=jnp.float32)
        + b2_ref[...], 0.0)
    out_ref[...] = (
        jnp.dot(h2, w3_ref[...], preferred_element_type=jnp.float32)
        + b3_ref[...])


def _mlp(ue, me, w1u, w1m, b1, w2, b2, w3, b3):
    fixed = lambda shape: pl.BlockSpec(shape, lambda i: (0, 0))
    return pl.pallas_call(
        _mlp_body,
        grid=(B // BLK,),
        in_specs=[
            pl.BlockSpec((BLK, 128), lambda i: (i, 0)),
            pl.BlockSpec((BLK, 128), lambda i: (i, 0)),
            fixed((D, 256)),
            fixed((D, 256)),
            fixed((1, 256)),
            fixed((256, 64)),
            fixed((1, 64)),
            fixed((64, 1)),
            fixed((1, 1)),
        ],
        out_specs=pl.BlockSpec((BLK, 1), lambda i: (i, 0)),
        out_shape=jax.ShapeDtypeStruct((B, 1), jnp.float32),
    )(ue, me, w1u, w1m, b1, w2, b2, w3, b3)


def kernel(user_id, movie_title, user_table, movie_table,
           W1, b1, W2, b2, W3, b3):
    ue, me = _sc_gather(user_table.T, movie_table.T,
                        user_id.astype(jnp.int32),
                        movie_title.astype(jnp.int32))
    return _mlp(ue, me, W1[:D], W1[D:], b1.reshape(1, 256),
                W2, b2.reshape(1, 64), W3, b3.reshape(1, 1))


# speedup vs baseline: 1.3663x; 1.3663x over previous
import functools

import jax
import jax.numpy as jnp
from jax import lax
from jax.experimental import pallas as pl
from jax.experimental.pallas import tpu as pltpu
from jax.experimental.pallas import tpu_sc as plsc

B = 16384
V = 100001
D = 32
NC, NS = 2, 16
NW = NC * NS
NBLK_TOT = 782
NBLK = 25
X = NBLK * 128
WIN = 2048
CPKCAP = WIN + 32
ROWCAP = 16
DUMP = B
OUTH = B + 8

_sc_mesh = plsc.VectorSubcoreMesh(core_axis_name="c", subcore_axis_name="s")


def _scan0(idxv, cpkA, cpkB, o0, o1, iota16):
    H = B // 2

    def group(k, carry):
        totA, totB = carry
        ivA = idxv[pl.ds(k * 16, 16)]
        ivB = idxv[pl.ds(H + k * 16, 16)]
        mA = (lax.shift_right_logical(ivA, 7) >= o0) &              (lax.shift_right_logical(ivA, 7) < o1)
        mB = (lax.shift_right_logical(ivB, 7) >= o0) &              (lax.shift_right_logical(ivB, 7) < o1)
        pkA = jnp.bitwise_or(lax.shift_left(ivA, 15), iota16 + k * 16)
        pkB = jnp.bitwise_or(lax.shift_left(ivB, 15), iota16 + (H + k * 16))
        plsc.store_compressed(
            cpkA.at[pl.ds(jnp.minimum(totA, WIN + 16), 16)], pkA, mask=mA)
        plsc.store_compressed(
            cpkB.at[pl.ds(jnp.minimum(totB, WIN + 16), 16)], pkB, mask=mB)
        totA = totA + plsc.all_reduce_population_count(mA)[0]
        totB = totB + plsc.all_reduce_population_count(mB)[0]
        return totA, totB

    totA, totB = lax.fori_loop(0, H // 16, group, (0, 0), unroll=4)
    return totA, totB


def _scan(idxv, cpk, o0, o1, iota16, win_lo, p0):
    def group(k, carry):
        tot, ca = carry
        iv = idxv[pl.ds(p0 + k * 16, 16)]
        blk = lax.shift_right_logical(iv, 7)
        m = (blk >= o0) & (blk < o1)
        pos = iota16 + (p0 + k * 16)
        pk = jnp.bitwise_or(lax.shift_left(iv, 15), pos)
        wok = (tot + 16 > win_lo) & (tot < win_lo + WIN)
        am = m & wok
        plsc.store_compressed(cpk.at[pl.ds(ca, 16)], pk, mask=am)
        tot = tot + plsc.all_reduce_population_count(m)[0]
        ca = ca + plsc.all_reduce_population_count(am)[0]
        return tot, ca
    return lax.fori_loop(0, B // 32, group, (0, 0), unroll=4)


def _extract(out_hbm, ca, sbase, vlin, cpk, rows_a, rows_b, posv,
             sem_s, iota16):
    nflush = (ca + (ROWCAP - 1)) // ROWCAP

    def flush(f, _):
        parity = f % 2

        def work(rows):
            @pl.when(f >= 2)
            def _():
                pltpu.make_async_copy(
                    out_hbm.at[pl.ds(0, ROWCAP)], rows, sem_s).wait()

            v0 = cpk[pl.ds(f * ROWCAP, 16)]
            il0 = jnp.clip(lax.shift_right_logical(v0, 15) - sbase, 0, X - 1)
            for e in range(ROWCAP):
                iloc = il0[e]
                cv = jnp.full((16,), iloc, jnp.int32)
                rows[e, pl.ds(0, 16)] = plsc.load_gather(vlin, [iota16, cv])
                rows[e, pl.ds(16, 16)] = plsc.load_gather(
                    vlin, [iota16 + 16, cv])
            gidx0 = f * ROWCAP + iota16
            posv[0, pl.ds(0, 16)] = jnp.where(
                gidx0 < ca, jnp.bitwise_and(v0, 32767), DUMP)
            pltpu.async_copy(rows, out_hbm.at[posv.at[0]], sem_s)

        @pl.when(parity == 0)
        def _():
            work(rows_a)

        @pl.when(parity == 1)
        def _():
            work(rows_b)

        return 0

    lax.fori_loop(0, nflush, flush, 0, unroll=False)

    @pl.when(nflush >= 1)
    def _():
        pltpu.make_async_copy(
            out_hbm.at[pl.ds(0, ROWCAP)], rows_a, sem_s).wait()

    @pl.when(nflush >= 2)
    def _():
        pltpu.make_async_copy(
            out_hbm.at[pl.ds(0, ROWCAP)], rows_b, sem_s).wait()


def _do_table(tT_hbm, idx_hbm, out_hbm, s_blk, o0, o1, vlin, idxv,
              cpkA, cpkB, rows_a, rows_b, posv, sem_t, sem_i, sem_s,
              iota16):
    stage = [
        pltpu.async_copy(
            tT_hbm.at[pl.ds(tj * 8, 8), pl.ds(s_blk * 128, X)],
            vlin.at[pl.ds(tj * 8, 8)], sem_t)
        for tj in range(4)
    ]
    pltpu.async_copy(idx_hbm, idxv, sem_i).wait()
    sbase = s_blk * 128

    totA, totB = _scan0(idxv, cpkA, cpkB, o0, o1, iota16)
    for s in stage:
        s.wait()
    _extract(out_hbm, jnp.minimum(totA, WIN), sbase, vlin, cpkA,
             rows_a, rows_b, posv, sem_s, iota16)
    _extract(out_hbm, jnp.minimum(totB, WIN), sbase, vlin, cpkB,
             rows_a, rows_b, posv, sem_s, iota16)

    for half, tot, p0 in ((cpkA, totA, 0), (cpkB, totB, B // 2)):
        def more(r, cpk_h=half, p0_h=p0):
            _, ca_r = _scan(idxv, cpk_h, o0, o1, iota16, r * WIN, p0_h)
            _extract(out_hbm, ca_r, sbase, vlin, cpk_h, rows_a, rows_b,
                     posv, sem_s, iota16)
            return r + 1

        lax.while_loop(lambda r, t=tot: r * WIN < t, more, 1)


@functools.partial(
    pl.kernel,
    out_type=(
        jax.ShapeDtypeStruct((OUTH, 128), jnp.float32),
        jax.ShapeDtypeStruct((OUTH, 128), jnp.float32),
    ),
    mesh=_sc_mesh,
    scratch_types=[
        pltpu.VMEM((D, X), jnp.float32),
        pltpu.VMEM((B,), jnp.int32),
        pltpu.VMEM((CPKCAP,), jnp.int32),
        pltpu.VMEM((CPKCAP,), jnp.int32),
        pltpu.VMEM((ROWCAP, 128), jnp.float32),
        pltpu.VMEM((ROWCAP, 128), jnp.float32),
        pltpu.VMEM((1, ROWCAP), jnp.int32),
        pltpu.SemaphoreType.DMA,
        pltpu.SemaphoreType.DMA,
        pltpu.SemaphoreType.DMA,
    ],
    compiler_params=pltpu.CompilerParams(needs_layout_passes=False),
)
def _sc_gather(uT_hbm, mT_hbm, uid_hbm, mid_hbm, ue_hbm, me_hbm,
               vlin, idxv, cpkA, cpkB, rows_a, rows_b, posv,
               sem_t, sem_i, sem_s):
    wid = lax.axis_index("s") * NC + lax.axis_index("c")
    o0 = wid * NBLK
    o1 = o0 + NBLK
    s_blk = jnp.minimum(o0, NBLK_TOT - NBLK)
    iota16 = lax.iota(jnp.int32, 16)

    _do_table(uT_hbm, uid_hbm, ue_hbm, s_blk, o0, o1, vlin, idxv,
              cpkA, cpkB, rows_a, rows_b, posv, sem_t, sem_i, sem_s, iota16)
    _do_table(mT_hbm, mid_hbm, me_hbm, s_blk, o0, o1, vlin, idxv,
              cpkA, cpkB, rows_a, rows_b, posv, sem_t, sem_i, sem_s, iota16)


BLK = 4096


def _mlp_body(ue_ref, me_ref, w1u_ref, w1m_ref, b1_ref, w2_ref, b2_ref,
              w3_ref, b3_ref, out_ref):
    ue = ue_ref[...][:, :D]
    me = me_ref[...][:, :D]
    x1 = jnp.dot(ue, w1u_ref[...], preferred_element_type=jnp.float32)
    x2 = jnp.dot(me, w1m_ref[...], preferred_element_type=jnp.float32)
    h1 = jnp.maximum(x1 + x2 + b1_ref[...], 0.0)
    h2 = jnp.maximum(
        jnp.dot(h1, w2_ref[...], preferred_element_type=jnp.float32)
        + b2_ref[...], 0.0)
    res = (jnp.dot(h2, w3_ref[...], preferred_element_type=jnp.float32)
           + b3_ref[...])
    out_ref[...] = res[:, 0]


def _mlp(ue, me, w1u, w1m, b1, w2, b2, w3, b3):
    fixed = lambda shape: pl.BlockSpec(shape, lambda i: (0, 0))
    return pl.pallas_call(
        _mlp_body,
        grid=(B // BLK,),
        in_specs=[
            pl.BlockSpec((BLK, 128), lambda i: (i, 0)),
            pl.BlockSpec((BLK, 128), lambda i: (i, 0)),
            fixed((D, 256)),
            fixed((D, 256)),
            fixed((1, 256)),
            fixed((256, 64)),
            fixed((1, 64)),
            fixed((64, 1)),
            fixed((1, 1)),
        ],
        out_specs=pl.BlockSpec((BLK,), lambda i: (i,)),
        out_shape=jax.ShapeDtypeStruct((B,), jnp.float32),
    )(ue, me, w1u, w1m, b1, w2, b2, w3, b3)


def kernel(user_id, movie_title, user_table, movie_table,
           W1, b1, W2, b2, W3, b3):
    ue, me = _sc_gather(user_table.T, movie_table.T,
                        user_id.astype(jnp.int32),
                        movie_title.astype(jnp.int32))
    return _mlp(ue, me, W1[:D], W1[D:], b1.reshape(1, 256),
                W2, b2.reshape(1, 64), W3, b3.reshape(1, 1)).reshape(B, 1)
